# baseline (speedup 1.0000x reference)
import dataclasses

import jax
import jax.numpy as jnp
from jax import lax
from jax.experimental import pallas as pl
from jax.experimental.pallas import tpu as pltpu
from jax.experimental.pallas import tpu_sc as plsc

N = 10000
D_IN = 128
D_HID = 64
N_CLASSES = 40

NC = 2
NS = 16
L = 16
K = 64
ACC_ROWS = 10112
ACC_COLS = 128
ROW_BLK = 1000

_E_TOTAL = 320000 + N
EP = ((_E_TOTAL + NC * NS * K - 1) // (NC * NS * K)) * (NC * NS * K)
CH = EP // (NC * NS * K)
PER_TILE = CH * K
ROWS_PER_SUB = ACC_ROWS // NS


def _norm_concat(h):
    nrm = jnp.sqrt(jnp.sum(h * h, axis=1, keepdims=True))
    g = h / jnp.maximum(nrm, 1e-12)
    return jnp.concatenate([g, h], axis=1)


def _proj_body(x_ref, w_ref, b_ref, gh_ref):
    h = jnp.maximum(x_ref[...] @ w_ref[...] + b_ref[...], 0.0)
    gh_ref[...] = _norm_concat(h)


def _combine_body(p_ref, gh_ref):
    s = p_ref[0] + p_ref[1]
    h = s[:, :D_HID] / s[:, D_HID:D_HID + 1]
    gh_ref[...] = _norm_concat(h)


def _head_body(p_ref, w_ref, b_ref, o_ref):
    s = p_ref[0] + p_ref[1]
    h = s[:, :D_HID] / s[:, D_HID:D_HID + 1]
    logits = h @ w_ref[...] + b_ref[...]
    m = jnp.max(logits, axis=1, keepdims=True)
    z = logits - m
    o_ref[...] = z - jnp.log(jnp.sum(jnp.exp(z), axis=1, keepdims=True))


_GRID = N // ROW_BLK

_proj = pl.pallas_call(
    _proj_body,
    grid=(_GRID,),
    in_specs=[
        pl.BlockSpec((ROW_BLK, D_IN), lambda i: (i, 0)),
        pl.BlockSpec((D_IN, D_HID), lambda i: (0, 0)),
        pl.BlockSpec((1, D_HID), lambda i: (0, 0)),
    ],
    out_specs=pl.BlockSpec((ROW_BLK, 2 * D_HID), lambda i: (i, 0)),
    out_shape=jax.ShapeDtypeStruct((N, 2 * D_HID), jnp.float32),
)

_combine = pl.pallas_call(
    _combine_body,
    grid=(_GRID,),
    in_specs=[pl.BlockSpec((NC, ROW_BLK, ACC_COLS), lambda i: (0, i, 0))],
    out_specs=pl.BlockSpec((ROW_BLK, 2 * D_HID), lambda i: (i, 0)),
    out_shape=jax.ShapeDtypeStruct((N, 2 * D_HID), jnp.float32),
)

_head = pl.pallas_call(
    _head_body,
    grid=(_GRID,),
    in_specs=[
        pl.BlockSpec((NC, ROW_BLK, ACC_COLS), lambda i: (0, i, 0)),
        pl.BlockSpec((D_HID, N_CLASSES), lambda i: (0, 0)),
        pl.BlockSpec((1, N_CLASSES), lambda i: (0, 0)),
    ],
    out_specs=pl.BlockSpec((ROW_BLK, N_CLASSES), lambda i: (i, 0)),
    out_shape=jax.ShapeDtypeStruct((N, N_CLASSES), jnp.float32),
)


def _edge_kernel_body(gh_hbm, src_hbm, dst_hbm, p_hbm,
                      acc_sh, idx_s, idx_d, gs, gd, wout, sem):
    core = lax.axis_index("c")
    sub = lax.axis_index("s")
    tid = core * NS + sub

    zeros16 = jnp.zeros((L,), jnp.float32)

    @pl.loop(0, K)
    def _(i):
        for j in range(ACC_COLS // L):
            wout[i, pl.ds(j * L, L)] = zeros16

    r0 = sub * ROWS_PER_SUB
    n_full, rem = ROWS_PER_SUB // K, ROWS_PER_SUB % K
    for t in range(n_full):
        pltpu.sync_copy(wout, acc_sh.at[pl.ds(r0 + t * K, K)])
    if rem:
        pltpu.sync_copy(wout.at[pl.ds(0, rem)],
                        acc_sh.at[pl.ds(r0 + n_full * K, rem)])

    plsc.subcore_barrier()

    tile_base = tid * PER_TILE

    @pl.loop(0, CH)
    def _(ch):
        base = tile_base + ch * K
        pltpu.sync_copy(src_hbm.at[pl.ds(base, K)], idx_s)
        pltpu.sync_copy(dst_hbm.at[pl.ds(base, K)], idx_d)
        c1 = pltpu.async_copy(gh_hbm.at[idx_s], gs, sem)
        c2 = pltpu.async_copy(gh_hbm.at[idx_d], gd, sem)
        c1.wait()
        c2.wait()

        for b in range(K // L):
            rows = lax.iota(jnp.int32, L) + (b * L)

            alpha = jnp.zeros((L,), jnp.float32)
            for d in range(D_HID):
                col = jnp.full((L,), d, jnp.int32)
                cs = plsc.load_gather(gs, [rows, col])
                cd = plsc.load_gather(gd, [rows, col])
                alpha = alpha + cs * cd
            w16 = jnp.exp(alpha)

            for d in range(D_HID):
                chs = plsc.load_gather(gs, [rows, jnp.full((L,), D_HID + d, jnp.int32)])
                plsc.store_scatter(wout, [rows, jnp.full((L,), d, jnp.int32)], chs * w16)

            plsc.store_scatter(wout, [rows, jnp.full((L,), D_HID, jnp.int32)], w16)

        pltpu.sync_copy(wout, acc_sh.at[idx_d], add=True)

    plsc.subcore_barrier()

    pltpu.sync_copy(acc_sh.at[pl.ds(r0, ROWS_PER_SUB)],
                    p_hbm.at[core, pl.ds(r0, ROWS_PER_SUB)])


_sc_params = pltpu.CompilerParams()
if "needs_layout_passes" in pltpu.CompilerParams.__dataclass_fields__:
    _sc_params = dataclasses.replace(_sc_params, needs_layout_passes=False)

_edge_kernel = pl.kernel(
    _edge_kernel_body,
    compiler_params=_sc_params,
    out_type=jax.ShapeDtypeStruct((NC, ACC_ROWS, ACC_COLS), jnp.float32),
    mesh=plsc.VectorSubcoreMesh(core_axis_name="c", subcore_axis_name="s"),
    scratch_types=[
        pltpu.VMEM_SHARED((ACC_ROWS, ACC_COLS), jnp.float32),
        pltpu.VMEM((K,), jnp.int32),
        pltpu.VMEM((K,), jnp.int32),
        pltpu.VMEM((K, ACC_COLS), jnp.float32),
        pltpu.VMEM((K, ACC_COLS), jnp.float32),
        pltpu.VMEM((K, ACC_COLS), jnp.float32),
        pltpu.SemaphoreType.DMA,
    ],
)


def kernel(x, edge_index, W1, b1, W2, b2):
    ar = jnp.arange(N, dtype=jnp.int32)
    src = jnp.concatenate([edge_index[0].astype(jnp.int32), ar])
    dst = jnp.concatenate([edge_index[1].astype(jnp.int32), ar])
    pad = EP - src.shape[0]
    src = jnp.concatenate([src, jnp.zeros((pad,), jnp.int32)])
    dst = jnp.concatenate([dst, jnp.full((pad,), N, jnp.int32)])

    gh = _proj(x, W1, b1.reshape(1, D_HID))
    for _ in range(3):
        p = _edge_kernel(gh, src, dst)
        gh = _combine(p)
    p = _edge_kernel(gh, src, dst)
    return _head(p, W2, b2.reshape(1, N_CLASSES))


# speedup vs baseline: 1.2295x; 1.2295x over previous
import dataclasses

import jax
import jax.numpy as jnp
from jax import lax
from jax.experimental import pallas as pl
from jax.experimental.pallas import tpu as pltpu
from jax.experimental.pallas import tpu_sc as plsc

N = 10000
D_IN = 128
D_HID = 64
N_CLASSES = 40

NC = 2
NS = 16
L = 16
K = 64
ACC_ROWS = 10112
ACC_COLS = 80
ROW_BLK = 1000

_E_TOTAL = 320000 + N
EP = ((_E_TOTAL + NC * NS * K - 1) // (NC * NS * K)) * (NC * NS * K)
CH = EP // (NC * NS * K)
PER_TILE = CH * K
ROWS_PER_SUB = ACC_ROWS // NS


def _norm_concat(h):
    nrm = jnp.sqrt(jnp.sum(h * h, axis=1, keepdims=True))
    g = h / jnp.maximum(nrm, 1e-12)
    return jnp.concatenate([g, h], axis=1)


def _proj_body(x_ref, w_ref, b_ref, gh_ref):
    h = jnp.maximum(x_ref[...] @ w_ref[...] + b_ref[...], 0.0)
    gh_ref[...] = _norm_concat(h)


def _combine_body(p_ref, gh_ref):
    s = p_ref[0] + p_ref[1]
    h = s[:, :D_HID] / s[:, D_HID:D_HID + 1]
    gh_ref[...] = _norm_concat(h)


def _head_body(p_ref, w_ref, b_ref, o_ref):
    s = p_ref[0] + p_ref[1]
    h = s[:, :D_HID] / s[:, D_HID:D_HID + 1]
    logits = h @ w_ref[...] + b_ref[...]
    m = jnp.max(logits, axis=1, keepdims=True)
    z = logits - m
    o_ref[...] = z - jnp.log(jnp.sum(jnp.exp(z), axis=1, keepdims=True))


_GRID = N // ROW_BLK

_proj = pl.pallas_call(
    _proj_body,
    grid=(_GRID,),
    in_specs=[
        pl.BlockSpec((ROW_BLK, D_IN), lambda i: (i, 0)),
        pl.BlockSpec((D_IN, D_HID), lambda i: (0, 0)),
        pl.BlockSpec((1, D_HID), lambda i: (0, 0)),
    ],
    out_specs=pl.BlockSpec((ROW_BLK, 2 * D_HID), lambda i: (i, 0)),
    out_shape=jax.ShapeDtypeStruct((N, 2 * D_HID), jnp.float32),
)

_combine = pl.pallas_call(
    _combine_body,
    grid=(_GRID,),
    in_specs=[pl.BlockSpec((NC, ROW_BLK, ACC_COLS), lambda i: (0, i, 0))],
    out_specs=pl.BlockSpec((ROW_BLK, 2 * D_HID), lambda i: (i, 0)),
    out_shape=jax.ShapeDtypeStruct((N, 2 * D_HID), jnp.float32),
)

_head = pl.pallas_call(
    _head_body,
    grid=(_GRID,),
    in_specs=[
        pl.BlockSpec((NC, ROW_BLK, ACC_COLS), lambda i: (0, i, 0)),
        pl.BlockSpec((D_HID, N_CLASSES), lambda i: (0, 0)),
        pl.BlockSpec((1, N_CLASSES), lambda i: (0, 0)),
    ],
    out_specs=pl.BlockSpec((ROW_BLK, N_CLASSES), lambda i: (i, 0)),
    out_shape=jax.ShapeDtypeStruct((N, N_CLASSES), jnp.float32),
)


def _edge_kernel_body(gh_hbm, src_hbm, dst_hbm, p_hbm, acc_sh,
                      is0, is1, id0, id1, ids0, ids1, gs0, gs1, gd0, gd1,
                      wo0, wo1, si0, si1, sg0, sg1, ss0, ss1):
    core = lax.axis_index("c")
    sub = lax.axis_index("s")
    tid = core * NS + sub

    IS, ID = (is0, is1), (id0, id1)
    IDS = (ids0, ids1)
    GS, GD = (gs0, gs1), (gd0, gd1)
    WO = (wo0, wo1)
    SI, SG, SS = (si0, si1), (sg0, sg1), (ss0, ss1)

    zeros16 = jnp.zeros((L,), jnp.float32)

    for wout in WO:
        @pl.loop(0, K)
        def _(i):
            for j in range(ACC_COLS // L):
                wout[i, pl.ds(j * L, L)] = zeros16

    r0 = sub * ROWS_PER_SUB
    n_full, rem = ROWS_PER_SUB // K, ROWS_PER_SUB % K
    for t in range(n_full):
        pltpu.sync_copy(WO[0], acc_sh.at[pl.ds(r0 + t * K, K)])
    if rem:
        pltpu.sync_copy(WO[0].at[pl.ds(0, rem)],
                        acc_sh.at[pl.ds(r0 + n_full * K, rem)])

    plsc.subcore_barrier()

    tile_base = tid * PER_TILE

    def start_idx(c, b):
        base = tile_base + c * K
        pltpu.async_copy(src_hbm.at[pl.ds(base, K)], IS[b], SI[b])
        pltpu.async_copy(dst_hbm.at[pl.ds(base, K)], ID[b], SI[b])

    def wait_idx(b):
        pltpu.make_async_copy(src_hbm.at[pl.ds(0, K)], IS[b], SI[b]).wait()
        pltpu.make_async_copy(dst_hbm.at[pl.ds(0, K)], ID[b], SI[b]).wait()

    def start_gather(b):
        pltpu.async_copy(gh_hbm.at[IS[b]], GS[b], SG[b])
        pltpu.async_copy(gh_hbm.at[ID[b]], GD[b], SG[b])

    def wait_gather(b):
        pltpu.make_async_copy(gh_hbm.at[IS[b]], GS[b], SG[b]).wait()
        pltpu.make_async_copy(gh_hbm.at[ID[b]], GD[b], SG[b]).wait()

    def wait_scatter(b):
        pltpu.make_async_copy(WO[b], acc_sh.at[IDS[b]], SS[b]).wait()

    start_idx(0, 0)
    wait_idx(0)
    start_gather(0)
    start_idx(1, 1)

    @pl.loop(0, CH, step=2)
    def _(c0):
        for b in range(2):
            c = c0 + b
            wait_idx(b ^ 1)
            start_gather(b ^ 1)
            wait_gather(b)

            @pl.when(c >= 2)
            def _():
                wait_scatter(b)

            gs, gd, wout = GS[b], GD[b], WO[b]
            for blk in range(K // L):
                rows = lax.iota(jnp.int32, L) + (blk * L)

                alpha = jnp.zeros((L,), jnp.float32)
                for d in range(D_HID):
                    col = jnp.full((L,), d, jnp.int32)
                    cs = plsc.load_gather(gs, [rows, col])
                    cd = plsc.load_gather(gd, [rows, col])
                    alpha = alpha + cs * cd
                w16 = jnp.exp(alpha)

                for d in range(D_HID):
                    chs = plsc.load_gather(gs, [rows, jnp.full((L,), D_HID + d, jnp.int32)])
                    plsc.store_scatter(wout, [rows, jnp.full((L,), d, jnp.int32)], chs * w16)

                plsc.store_scatter(wout, [rows, jnp.full((L,), D_HID, jnp.int32)], w16)

            for j in range(K // L):
                IDS[b][pl.ds(j * L, L)] = ID[b][pl.ds(j * L, L)]
            pltpu.async_copy(wout, acc_sh.at[IDS[b]], SS[b], add=True)
            start_idx(c + 2, b)

    wait_idx(1)
    wait_gather(0)
    wait_scatter(0)
    wait_scatter(1)

    plsc.subcore_barrier()

    pltpu.sync_copy(acc_sh.at[pl.ds(r0, ROWS_PER_SUB)],
                    p_hbm.at[core, pl.ds(r0, ROWS_PER_SUB)])


_sc_params = pltpu.CompilerParams()
if "needs_layout_passes" in pltpu.CompilerParams.__dataclass_fields__:
    _sc_params = dataclasses.replace(_sc_params, needs_layout_passes=False)

_edge_kernel = pl.kernel(
    _edge_kernel_body,
    compiler_params=_sc_params,
    out_type=jax.ShapeDtypeStruct((NC, ACC_ROWS, ACC_COLS), jnp.float32),
    mesh=plsc.VectorSubcoreMesh(core_axis_name="c", subcore_axis_name="s"),
    scratch_types=[
        pltpu.VMEM_SHARED((ACC_ROWS, ACC_COLS), jnp.float32),
        pltpu.VMEM((K,), jnp.int32),
        pltpu.VMEM((K,), jnp.int32),
        pltpu.VMEM((K,), jnp.int32),
        pltpu.VMEM((K,), jnp.int32),
        pltpu.VMEM((K,), jnp.int32),
        pltpu.VMEM((K,), jnp.int32),
        pltpu.VMEM((K, 2 * D_HID), jnp.float32),
        pltpu.VMEM((K, 2 * D_HID), jnp.float32),
        pltpu.VMEM((K, 2 * D_HID), jnp.float32),
        pltpu.VMEM((K, 2 * D_HID), jnp.float32),
        pltpu.VMEM((K, ACC_COLS), jnp.float32),
        pltpu.VMEM((K, ACC_COLS), jnp.float32),
        pltpu.SemaphoreType.DMA,
        pltpu.SemaphoreType.DMA,
        pltpu.SemaphoreType.DMA,
        pltpu.SemaphoreType.DMA,
        pltpu.SemaphoreType.DMA,
        pltpu.SemaphoreType.DMA,
    ],
)


def kernel(x, edge_index, W1, b1, W2, b2):
    ar = jnp.arange(N, dtype=jnp.int32)
    src = jnp.concatenate([edge_index[0].astype(jnp.int32), ar])
    dst = jnp.concatenate([edge_index[1].astype(jnp.int32), ar])
    pad = EP + 2 * K - src.shape[0]
    src = jnp.concatenate([src, jnp.zeros((pad,), jnp.int32)])
    dst = jnp.concatenate([dst, jnp.full((pad,), N, jnp.int32)])

    gh = _proj(x, W1, b1.reshape(1, D_HID))
    for _ in range(3):
        p = _edge_kernel(gh, src, dst)
        gh = _combine(p)
    p = _edge_kernel(gh, src, dst)
    return _head(p, W2, b2.reshape(1, N_CLASSES))


# speedup vs baseline: 5.1435x; 4.1834x over previous
import dataclasses

import jax
import jax.numpy as jnp
from jax import lax
from jax.experimental import pallas as pl
from jax.experimental.pallas import tpu as pltpu
from jax.experimental.pallas import tpu_sc as plsc

N = 10000
D_IN = 128
D_HID = 64
N_CLASSES = 40

NC = 2
NS = 16
L = 16
K = 64
ACC_ROWS = 10112
ACC_COLS = 80
ROW_BLK = 1000

_E_TOTAL = 320000 + N
EP = ((_E_TOTAL + NC * NS * K - 1) // (NC * NS * K)) * (NC * NS * K)
CH = EP // (NC * NS * K)
PER_TILE = CH * K
ROWS_PER_SUB = ACC_ROWS // NS


def _norm_concat(h):
    nrm = jnp.sqrt(jnp.sum(h * h, axis=1, keepdims=True))
    g = h / jnp.maximum(nrm, 1e-12)
    return jnp.concatenate([g, h], axis=1)


def _proj_body(x_ref, w_ref, b_ref, gh_ref):
    h = jnp.maximum(x_ref[...] @ w_ref[...] + b_ref[...], 0.0)
    gh_ref[...] = _norm_concat(h)


def _combine_body(p_ref, gh_ref):
    s = p_ref[0] + p_ref[1]
    h = s[:, :D_HID] / s[:, D_HID:D_HID + 1]
    gh_ref[...] = _norm_concat(h)


def _head_body(p_ref, w_ref, b_ref, o_ref):
    s = p_ref[0] + p_ref[1]
    h = s[:, :D_HID] / s[:, D_HID:D_HID + 1]
    logits = h @ w_ref[...] + b_ref[...]
    m = jnp.max(logits, axis=1, keepdims=True)
    z = logits - m
    o_ref[...] = z - jnp.log(jnp.sum(jnp.exp(z), axis=1, keepdims=True))


_GRID = N // ROW_BLK

_proj = pl.pallas_call(
    _proj_body,
    grid=(_GRID,),
    in_specs=[
        pl.BlockSpec((ROW_BLK, D_IN), lambda i: (i, 0)),
        pl.BlockSpec((D_IN, D_HID), lambda i: (0, 0)),
        pl.BlockSpec((1, D_HID), lambda i: (0, 0)),
    ],
    out_specs=pl.BlockSpec((ROW_BLK, 2 * D_HID), lambda i: (i, 0)),
    out_shape=jax.ShapeDtypeStruct((N, 2 * D_HID), jnp.float32),
)

_combine = pl.pallas_call(
    _combine_body,
    grid=(_GRID,),
    in_specs=[pl.BlockSpec((NC, ROW_BLK, ACC_COLS), lambda i: (0, i, 0))],
    out_specs=pl.BlockSpec((ROW_BLK, 2 * D_HID), lambda i: (i, 0)),
    out_shape=jax.ShapeDtypeStruct((N, 2 * D_HID), jnp.float32),
)

_head = pl.pallas_call(
    _head_body,
    grid=(_GRID,),
    in_specs=[
        pl.BlockSpec((NC, ROW_BLK, ACC_COLS), lambda i: (0, i, 0)),
        pl.BlockSpec((D_HID, N_CLASSES), lambda i: (0, 0)),
        pl.BlockSpec((1, N_CLASSES), lambda i: (0, 0)),
    ],
    out_specs=pl.BlockSpec((ROW_BLK, N_CLASSES), lambda i: (i, 0)),
    out_shape=jax.ShapeDtypeStruct((N, N_CLASSES), jnp.float32),
)


def _edge_kernel_body(gh_hbm, src_hbm, dst_hbm, p_hbm, acc_sh,
                      is0, is1, id0, id1, ids0, ids1, gs0, gs1, gd0, gd1,
                      wo0, wo1, si0, si1, sg0, sg1, ss0, ss1):
    core = lax.axis_index("c")
    sub = lax.axis_index("s")
    tid = core * NS + sub

    IS, ID = (is0, is1), (id0, id1)
    IDS = (ids0, ids1)
    GS, GD = (gs0, gs1), (gd0, gd1)
    WO = (wo0, wo1)
    SI, SG, SS = (si0, si1), (sg0, sg1), (ss0, ss1)

    zeros16 = jnp.zeros((L,), jnp.float32)

    for wout in WO:
        @pl.loop(0, K)
        def _(i):
            for j in range(ACC_COLS // L):
                wout[i, pl.ds(j * L, L)] = zeros16

    r0 = sub * ROWS_PER_SUB
    n_full, rem = ROWS_PER_SUB // K, ROWS_PER_SUB % K
    for t in range(n_full):
        pltpu.sync_copy(WO[0], acc_sh.at[pl.ds(r0 + t * K, K)])
    if rem:
        pltpu.sync_copy(WO[0].at[pl.ds(0, rem)],
                        acc_sh.at[pl.ds(r0 + n_full * K, rem)])

    plsc.subcore_barrier()

    tile_base = tid * PER_TILE

    def start_idx(c, b):
        base = tile_base + c * K
        pltpu.async_copy(src_hbm.at[pl.ds(base, K)], IS[b], SI[b])
        pltpu.async_copy(dst_hbm.at[pl.ds(base, K)], ID[b], SI[b])

    def wait_idx(b):
        pltpu.make_async_copy(src_hbm.at[pl.ds(0, K)], IS[b], SI[b]).wait()
        pltpu.make_async_copy(dst_hbm.at[pl.ds(0, K)], ID[b], SI[b]).wait()

    def start_gather(b):
        pltpu.async_copy(gh_hbm.at[IS[b]], GS[b], SG[b])
        pltpu.async_copy(gh_hbm.at[ID[b]], GD[b], SG[b])

    def wait_gather(b):
        pltpu.make_async_copy(gh_hbm.at[IS[b]], GS[b], SG[b]).wait()
        pltpu.make_async_copy(gh_hbm.at[ID[b]], GD[b], SG[b]).wait()

    def wait_scatter(b):
        pltpu.make_async_copy(WO[b], acc_sh.at[IDS[b]], SS[b]).wait()

    start_idx(0, 0)
    wait_idx(0)
    start_gather(0)
    start_idx(1, 1)

    @pl.loop(0, CH, step=2)
    def _(c0):
        for b in range(2):
            c = c0 + b
            wait_idx(b ^ 1)
            start_gather(b ^ 1)
            wait_gather(b)

            @pl.when(c >= 2)
            def _():
                wait_scatter(b)

            gs, gd, wout = GS[b], GD[b], WO[b]
            for e in range(K):
                acc = (gs[e, pl.ds(0, L)] * gd[e, pl.ds(0, L)]
                       + gs[e, pl.ds(L, L)] * gd[e, pl.ds(L, L)]
                       + gs[e, pl.ds(2 * L, L)] * gd[e, pl.ds(2 * L, L)]
                       + gs[e, pl.ds(3 * L, L)] * gd[e, pl.ds(3 * L, L)])
                alpha = jnp.sum(acc)
                w = jnp.exp(jnp.full((L,), alpha))
                wout[e, pl.ds(0, L)] = gs[e, pl.ds(4 * L, L)] * w
                wout[e, pl.ds(L, L)] = gs[e, pl.ds(5 * L, L)] * w
                wout[e, pl.ds(2 * L, L)] = gs[e, pl.ds(6 * L, L)] * w
                wout[e, pl.ds(3 * L, L)] = gs[e, pl.ds(7 * L, L)] * w
                wout[e, pl.ds(4 * L, L)] = w

            for j in range(K // L):
                IDS[b][pl.ds(j * L, L)] = ID[b][pl.ds(j * L, L)]
            pltpu.async_copy(wout, acc_sh.at[IDS[b]], SS[b], add=True)
            start_idx(c + 2, b)

    wait_idx(1)
    wait_gather(0)
    wait_scatter(0)
    wait_scatter(1)

    plsc.subcore_barrier()

    pltpu.sync_copy(acc_sh.at[pl.ds(r0, ROWS_PER_SUB)],
                    p_hbm.at[core, pl.ds(r0, ROWS_PER_SUB)])


_sc_params = pltpu.CompilerParams()
if "needs_layout_passes" in pltpu.CompilerParams.__dataclass_fields__:
    _sc_params = dataclasses.replace(_sc_params, needs_layout_passes=False)

_edge_kernel = pl.kernel(
    _edge_kernel_body,
    compiler_params=_sc_params,
    out_type=jax.ShapeDtypeStruct((NC, ACC_ROWS, ACC_COLS), jnp.float32),
    mesh=plsc.VectorSubcoreMesh(core_axis_name="c", subcore_axis_name="s"),
    scratch_types=[
        pltpu.VMEM_SHARED((ACC_ROWS, ACC_COLS), jnp.float32),
        pltpu.VMEM((K,), jnp.int32),
        pltpu.VMEM((K,), jnp.int32),
        pltpu.VMEM((K,), jnp.int32),
        pltpu.VMEM((K,), jnp.int32),
        pltpu.VMEM((K,), jnp.int32),
        pltpu.VMEM((K,), jnp.int32),
        pltpu.VMEM((K, 2 * D_HID), jnp.float32),
        pltpu.VMEM((K, 2 * D_HID), jnp.float32),
        pltpu.VMEM((K, 2 * D_HID), jnp.float32),
        pltpu.VMEM((K, 2 * D_HID), jnp.float32),
        pltpu.VMEM((K, ACC_COLS), jnp.float32),
        pltpu.VMEM((K, ACC_COLS), jnp.float32),
        pltpu.SemaphoreType.DMA,
        pltpu.SemaphoreType.DMA,
        pltpu.SemaphoreType.DMA,
        pltpu.SemaphoreType.DMA,
        pltpu.SemaphoreType.DMA,
        pltpu.SemaphoreType.DMA,
    ],
)


def kernel(x, edge_index, W1, b1, W2, b2):
    ar = jnp.arange(N, dtype=jnp.int32)
    src = jnp.concatenate([edge_index[0].astype(jnp.int32), ar])
    dst = jnp.concatenate([edge_index[1].astype(jnp.int32), ar])
    pad = EP + 2 * K - src.shape[0]
    src = jnp.concatenate([src, jnp.zeros((pad,), jnp.int32)])
    dst = jnp.concatenate([dst, jnp.full((pad,), N, jnp.int32)])

    gh = _proj(x, W1, b1.reshape(1, D_HID))
    for _ in range(3):
        p = _edge_kernel(gh, src, dst)
        gh = _combine(p)
    p = _edge_kernel(gh, src, dst)
    return _head(p, W2, b2.reshape(1, N_CLASSES))


# speedup vs baseline: 5.4015x; 1.0502x over previous
import dataclasses

import jax
import jax.numpy as jnp
from jax import lax
from jax.experimental import pallas as pl
from jax.experimental.pallas import tpu as pltpu
from jax.experimental.pallas import tpu_sc as plsc

N = 10000
D_IN = 128
D_HID = 64
N_CLASSES = 40

NC = 2
NS = 16
L = 16
K = 64
ACC_ROWS = 10112
ACC_COLS = 80
ROW_BLK = 1000

_E_TOTAL = 320000 + N
EP = ((_E_TOTAL + NC * NS * K - 1) // (NC * NS * K)) * (NC * NS * K)
CH = EP // (NC * NS * K)
PER_TILE = CH * K
ROWS_PER_SUB = ACC_ROWS // NS


def _norm_concat(h):
    nrm = jnp.sqrt(jnp.sum(h * h, axis=1, keepdims=True))
    g = h / jnp.maximum(nrm, 1e-12)
    return jnp.concatenate([g, h], axis=1)


def _proj_body(x_ref, w_ref, b_ref, gh_ref):
    h = jnp.maximum(x_ref[...] @ w_ref[...] + b_ref[...], 0.0)
    gh_ref[...] = _norm_concat(h)


def _combine_body(p_ref, gh_ref):
    s = p_ref[0] + p_ref[1]
    h = s[:, :D_HID] / s[:, D_HID:D_HID + 1]
    gh_ref[...] = _norm_concat(h)


def _head_body(p_ref, w_ref, b_ref, o_ref):
    s = p_ref[0] + p_ref[1]
    h = s[:, :D_HID] / s[:, D_HID:D_HID + 1]
    logits = h @ w_ref[...] + b_ref[...]
    m = jnp.max(logits, axis=1, keepdims=True)
    z = logits - m
    o_ref[...] = z - jnp.log(jnp.sum(jnp.exp(z), axis=1, keepdims=True))


_GRID = N // ROW_BLK

_proj = pl.pallas_call(
    _proj_body,
    grid=(_GRID,),
    in_specs=[
        pl.BlockSpec((ROW_BLK, D_IN), lambda i: (i, 0)),
        pl.BlockSpec((D_IN, D_HID), lambda i: (0, 0)),
        pl.BlockSpec((1, D_HID), lambda i: (0, 0)),
    ],
    out_specs=pl.BlockSpec((ROW_BLK, 2 * D_HID), lambda i: (i, 0)),
    out_shape=jax.ShapeDtypeStruct((N, 2 * D_HID), jnp.float32),
)

_combine = pl.pallas_call(
    _combine_body,
    grid=(_GRID,),
    in_specs=[pl.BlockSpec((NC, ROW_BLK, ACC_COLS), lambda i: (0, i, 0))],
    out_specs=pl.BlockSpec((ROW_BLK, 2 * D_HID), lambda i: (i, 0)),
    out_shape=jax.ShapeDtypeStruct((N, 2 * D_HID), jnp.float32),
)

_head = pl.pallas_call(
    _head_body,
    grid=(_GRID,),
    in_specs=[
        pl.BlockSpec((NC, ROW_BLK, ACC_COLS), lambda i: (0, i, 0)),
        pl.BlockSpec((D_HID, N_CLASSES), lambda i: (0, 0)),
        pl.BlockSpec((1, N_CLASSES), lambda i: (0, 0)),
    ],
    out_specs=pl.BlockSpec((ROW_BLK, N_CLASSES), lambda i: (i, 0)),
    out_shape=jax.ShapeDtypeStruct((N, N_CLASSES), jnp.float32),
)


def _edge_kernel_body(gh_hbm, src_hbm, dst_hbm, p_hbm, acc_sh,
                      is0, is1, id0, id1, ids0, ids1, gs0, gs1, gd0, gd1,
                      wo0, wo1, si0, si1, sg0, sg1, ss0, ss1):
    core = lax.axis_index("c")
    sub = lax.axis_index("s")
    tid = core * NS + sub

    IS, ID = (is0, is1), (id0, id1)
    IDS = (ids0, ids1)
    GS, GD = (gs0, gs1), (gd0, gd1)
    WO = (wo0, wo1)
    SI, SG, SS = (si0, si1), (sg0, sg1), (ss0, ss1)

    zeros16 = jnp.zeros((L,), jnp.float32)

    for wout in WO:
        @pl.loop(0, K)
        def _(i):
            for j in range(ACC_COLS // L):
                wout[i, pl.ds(j * L, L)] = zeros16

    r0 = sub * ROWS_PER_SUB
    n_full, rem = ROWS_PER_SUB // K, ROWS_PER_SUB % K
    for t in range(n_full):
        pltpu.sync_copy(WO[0], acc_sh.at[pl.ds(r0 + t * K, K)])
    if rem:
        pltpu.sync_copy(WO[0].at[pl.ds(0, rem)],
                        acc_sh.at[pl.ds(r0 + n_full * K, rem)])

    plsc.subcore_barrier()

    tile_base = tid * PER_TILE
    lane = lax.iota(jnp.int32, L)
    perms = [lane ^ k for k in (1, 2, 4, 8)]

    def start_idx(c, b):
        base = tile_base + c * K
        pltpu.async_copy(src_hbm.at[pl.ds(base, K)], IS[b], SI[b])
        pltpu.async_copy(dst_hbm.at[pl.ds(base, K)], ID[b], SI[b])

    def wait_idx(b):
        pltpu.make_async_copy(src_hbm.at[pl.ds(0, K)], IS[b], SI[b]).wait()
        pltpu.make_async_copy(dst_hbm.at[pl.ds(0, K)], ID[b], SI[b]).wait()

    def start_gather(b):
        pltpu.async_copy(gh_hbm.at[IS[b]], GS[b], SG[b])
        pltpu.async_copy(gh_hbm.at[ID[b]], GD[b], SG[b])

    def wait_gather(b):
        pltpu.make_async_copy(gh_hbm.at[IS[b]], GS[b], SG[b]).wait()
        pltpu.make_async_copy(gh_hbm.at[ID[b]], GD[b], SG[b]).wait()

    def wait_scatter(b):
        pltpu.make_async_copy(WO[b], acc_sh.at[IDS[b]], SS[b]).wait()

    start_idx(0, 0)
    wait_idx(0)
    start_gather(0)
    start_idx(1, 1)

    @pl.loop(0, CH, step=2)
    def _(c0):
        for b in range(2):
            c = c0 + b
            wait_idx(b ^ 1)
            start_gather(b ^ 1)
            wait_gather(b)

            @pl.when(c >= 2)
            def _():
                wait_scatter(b)

            gs, gd, wout = GS[b], GD[b], WO[b]
            for e in range(K):
                acc = (gs[e, pl.ds(0, L)] * gd[e, pl.ds(0, L)]
                       + gs[e, pl.ds(L, L)] * gd[e, pl.ds(L, L)]
                       + gs[e, pl.ds(2 * L, L)] * gd[e, pl.ds(2 * L, L)]
                       + gs[e, pl.ds(3 * L, L)] * gd[e, pl.ds(3 * L, L)])
                for p in perms:
                    acc = acc + acc.at[p].get(mode="promise_in_bounds")
                w = jnp.exp(acc)
                wout[e, pl.ds(0, L)] = gs[e, pl.ds(4 * L, L)] * w
                wout[e, pl.ds(L, L)] = gs[e, pl.ds(5 * L, L)] * w
                wout[e, pl.ds(2 * L, L)] = gs[e, pl.ds(6 * L, L)] * w
                wout[e, pl.ds(3 * L, L)] = gs[e, pl.ds(7 * L, L)] * w
                wout[e, pl.ds(4 * L, L)] = w

            for j in range(K // L):
                IDS[b][pl.ds(j * L, L)] = ID[b][pl.ds(j * L, L)]
            pltpu.async_copy(wout, acc_sh.at[IDS[b]], SS[b], add=True)
            start_idx(c + 2, b)

    wait_idx(1)
    wait_gather(0)
    wait_scatter(0)
    wait_scatter(1)

    plsc.subcore_barrier()

    pltpu.sync_copy(acc_sh.at[pl.ds(r0, ROWS_PER_SUB)],
                    p_hbm.at[core, pl.ds(r0, ROWS_PER_SUB)])


_sc_params = pltpu.CompilerParams()
if "needs_layout_passes" in pltpu.CompilerParams.__dataclass_fields__:
    _sc_params = dataclasses.replace(_sc_params, needs_layout_passes=False)

_edge_kernel = pl.kernel(
    _edge_kernel_body,
    compiler_params=_sc_params,
    out_type=jax.ShapeDtypeStruct((NC, ACC_ROWS, ACC_COLS), jnp.float32),
    mesh=plsc.VectorSubcoreMesh(core_axis_name="c", subcore_axis_name="s"),
    scratch_types=[
        pltpu.VMEM_SHARED((ACC_ROWS, ACC_COLS), jnp.float32),
        pltpu.VMEM((K,), jnp.int32),
        pltpu.VMEM((K,), jnp.int32),
        pltpu.VMEM((K,), jnp.int32),
        pltpu.VMEM((K,), jnp.int32),
        pltpu.VMEM((K,), jnp.int32),
        pltpu.VMEM((K,), jnp.int32),
        pltpu.VMEM((K, 2 * D_HID), jnp.float32),
        pltpu.VMEM((K, 2 * D_HID), jnp.float32),
        pltpu.VMEM((K, 2 * D_HID), jnp.float32),
        pltpu.VMEM((K, 2 * D_HID), jnp.float32),
        pltpu.VMEM((K, ACC_COLS), jnp.float32),
        pltpu.VMEM((K, ACC_COLS), jnp.float32),
        pltpu.SemaphoreType.DMA,
        pltpu.SemaphoreType.DMA,
        pltpu.SemaphoreType.DMA,
        pltpu.SemaphoreType.DMA,
        pltpu.SemaphoreType.DMA,
        pltpu.SemaphoreType.DMA,
    ],
)


def kernel(x, edge_index, W1, b1, W2, b2):
    ar = jnp.arange(N, dtype=jnp.int32)
    src = jnp.concatenate([edge_index[0].astype(jnp.int32), ar])
    dst = jnp.concatenate([edge_index[1].astype(jnp.int32), ar])
    pad = EP + 2 * K - src.shape[0]
    src = jnp.concatenate([src, jnp.zeros((pad,), jnp.int32)])
    dst = jnp.concatenate([dst, jnp.full((pad,), N, jnp.int32)])

    gh = _proj(x, W1, b1.reshape(1, D_HID))
    for _ in range(3):
        p = _edge_kernel(gh, src, dst)
        gh = _combine(p)
    p = _edge_kernel(gh, src, dst)
    return _head(p, W2, b2.reshape(1, N_CLASSES))
